# baseline (speedup 1.0000x reference)
import functools

import jax
import jax.numpy as jnp
from jax import lax
from jax.experimental import pallas as pl
from jax.experimental.pallas import tpu as pltpu
from jax.experimental.pallas import tpu_sc as plsc

RES_ = 128
GRID_ = RES_ * RES_ * RES_
NC_ = 2
NS_ = 16
LANES_ = 16
HALF_ = GRID_ // NC_
TILEG_ = HALF_ // NS_
P_ = 1024
NVEC_ = P_ // LANES_
RP_ = NS_ * P_
RV_ = RP_ // LANES_
QD_ = 4


def _voxelize(p):
    u = (p + 1.0) * jnp.float32(0.5) * jnp.float32(RES_)
    i = u.astype(jnp.int32)
    return jnp.clip(i, 0, RES_ - 1)


def _sc_body(n_total, pos_hbm, imp_hbm, gv_hbm, out_hbm, idxscr_hbm,
             posbuf, prodidx, considx, consval2, resp4, outbuf,
             subgrid,
             sp_resp, sem_pos, sem_scr, sem_ci, sem_cv, sem_resp):
    c = lax.axis_index("c")
    s = lax.axis_index("s")
    chunk = n_total // NS_
    rounds = chunk // P_
    half_lo = c * HALF_
    lane = jnp.arange(LANES_, dtype=jnp.int32)
    lane3 = lane * 3

    pltpu.sync_copy(gv_hbm.at[pl.ds(half_lo + s * TILEG_, TILEG_)],
                    subgrid.at[pl.ds(0, TILEG_)])
    subgrid[pl.ds(TILEG_, LANES_)] = jnp.zeros((LANES_,), jnp.float32)
    plsc.subcore_barrier()

    chunkbase = s * chunk

    def pos_copy(r, pb):
        return pltpu.make_async_copy(
            pos_hbm.at[pl.ds((chunkbase + r * P_) * 3, P_ * 3)],
            posbuf.at[pl.ds(pb * (P_ * 3), P_ * 3)], sem_pos)

    def produce(r, pb):
        pos_copy(r, pb).wait()

        @pl.loop(0, NVEC_, unroll=8)
        def _vec(j):
            off = pb * (P_ * 3) + j * (3 * LANES_)
            x = plsc.load_gather(posbuf, [lane3 + off])
            y = plsc.load_gather(posbuf, [lane3 + (off + 1)])
            z = plsc.load_gather(posbuf, [lane3 + (off + 2)])
            lin = (_voxelize(x) * RES_ + _voxelize(y)) * RES_ + _voxelize(z)
            keep = (lin >= half_lo) & (lin < half_lo + HALF_)
            prodidx[pl.ds(j * LANES_, LANES_)] = (
                jnp.where(keep, lin - half_lo, HALF_))

        cp_scr = pltpu.async_copy(
            prodidx, idxscr_hbm.at[c, pl.ds(r * RP_ + s * P_, P_)], sem_scr)

        @pl.when(r + 1 < rounds)
        def _prefetch():
            pos_copy(r + 1, 1 - pb).start()
        cp_scr.wait()

    pos_copy(0, 0).start()

    @pl.loop(0, rounds)
    def _p1(r):
        produce(r, r % 2)
        plsc.subcore_barrier()

        cp_ci = pltpu.async_copy(idxscr_hbm.at[c, pl.ds(r * RP_, RP_)],
                                 considx, sem_ci)
        cp_cv = pltpu.async_copy(imp_hbm.at[:, pl.ds(r * P_, P_)],
                                 consval2, sem_cv)
        cp_ci.wait()
        cp_cv.wait()

        @pl.loop(0, NS_)
        def _addp(p):
            @pl.loop(0, NVEC_, unroll=8)
            def _add(j):
                vidx = considx[pl.ds(p * P_ + j * LANES_, LANES_)]
                vval = consval2[p, pl.ds(j * LANES_, LANES_)]
                own = (vidx >> 16) == s
                plsc.addupdate_scatter(subgrid, [vidx & (TILEG_ - 1)], vval,
                                       mask=own)
        plsc.subcore_barrier()

    @pl.loop(0, rounds)
    def _p2(r):
        base = chunkbase + r * P_
        pltpu.sync_copy(idxscr_hbm.at[c, pl.ds(r * RP_, RP_)], considx)

        @pl.loop(0, NS_)
        def _ansp(p):
            @pl.loop(0, NVEC_, unroll=8)
            def _ans(j):
                vidx = considx[pl.ds(p * P_ + j * LANES_, LANES_)]
                own = (vidx >> 16) == s
                lidx = jnp.where(own, vidx & (TILEG_ - 1), TILEG_)
                consval2[p, pl.ds(j * LANES_, LANES_)] = (
                    plsc.load_gather(subgrid, [lidx]))

        copies = [pltpu.async_copy(
            consval2.at[p],
            sp_resp.at[pl.ds((p * NS_ + s) * P_, P_)], sem_resp)
            for p in range(NS_)]
        for cp in copies:
            cp.wait()
        plsc.subcore_barrier()

        for q in range(NS_ // QD_):
            pltpu.sync_copy(
                sp_resp.at[pl.ds((s * NS_ + q * QD_) * P_, QD_ * P_)],
                resp4.at[pl.ds(0, QD_ * P_)])

            @pl.loop(0, NVEC_, unroll=8)
            def _comb(j):
                slot = lane + j * LANES_
                vidx = considx[pl.ds(s * P_ + j * LANES_, LANES_)]
                dloc = (vidx >> 16) - q * QD_
                inh = (dloc >= 0) & (dloc < QD_)
                rv = plsc.load_gather(
                    resp4, [jnp.where(inh, dloc, QD_) * P_ + slot])
                val = jnp.where(inh, rv, jnp.float32(0.0))
                if q == 0:
                    outbuf[pl.ds(j * LANES_, LANES_)] = val
                else:
                    prev = outbuf[pl.ds(j * LANES_, LANES_)]
                    outbuf[pl.ds(j * LANES_, LANES_)] = prev + val

        pltpu.sync_copy(outbuf, out_hbm.at[c, pl.ds(base, P_)])
        plsc.subcore_barrier()


def kernel(positions, importances, grid_values):
    n = positions.shape[0]
    pos_flat = positions.reshape(-1)
    imp2d = importances.reshape(NS_, n // NS_)
    gv = grid_values.reshape(-1)
    mesh = plsc.VectorSubcoreMesh(core_axis_name="c", subcore_axis_name="s",
                                  num_cores=NC_, num_subcores=NS_)
    out, _ = pl.kernel(
        functools.partial(_sc_body, n),
        out_type=(jax.ShapeDtypeStruct((NC_, n), jnp.float32),
                  jax.ShapeDtypeStruct((NC_, n), jnp.int32)),
        mesh=mesh,
        compiler_params=pltpu.CompilerParams(needs_layout_passes=False),
        scratch_types=[
            pltpu.VMEM((2 * P_ * 3,), jnp.float32),
            pltpu.VMEM((P_,), jnp.int32),
            pltpu.VMEM((RP_,), jnp.int32),
            pltpu.VMEM((NS_, P_), jnp.float32),
            pltpu.VMEM(((QD_ + 1) * P_,), jnp.float32),
            pltpu.VMEM((P_,), jnp.float32),
            pltpu.VMEM((TILEG_ + LANES_,), jnp.float32),
            pltpu.VMEM_SHARED((NS_ * RP_,), jnp.float32),
            pltpu.SemaphoreType.DMA,
            pltpu.SemaphoreType.DMA,
            pltpu.SemaphoreType.DMA,
            pltpu.SemaphoreType.DMA,
            pltpu.SemaphoreType.DMA,
        ],
    )(pos_flat, imp2d, gv)
    return (out[0] + out[1]).reshape(n, 1)


# speedup vs baseline: 1.6944x; 1.6944x over previous
import functools

import jax
import jax.numpy as jnp
from jax import lax
from jax.experimental import pallas as pl
from jax.experimental.pallas import tpu as pltpu
from jax.experimental.pallas import tpu_sc as plsc

RES_ = 128
GRID_ = RES_ * RES_ * RES_
NC_ = 2
NS_ = 16
LANES_ = 16
HALF_ = GRID_ // NC_
TILEG_ = HALF_ // NS_
P_ = 1024
NVEC_ = P_ // LANES_
RP_ = NS_ * P_
HB_ = RP_ // 2
HV_ = HB_ // LANES_


def _voxelize(p):
    u = (p + 1.0) * jnp.float32(0.5) * jnp.float32(RES_)
    i = u.astype(jnp.int32)
    return jnp.clip(i, 0, RES_ - 1)


def _sc_body(n_total, pos_hbm, imp_hbm, gv_hbm, out_hbm, idxscr_hbm,
             posbuf, prodidx, considx, consval, resp8, outbuf, subgrid,
             sp_idx, sp_val, sp_resp,
             sem_pos, sem_imp, sem_scr, sem_ci, sem_cv, sem_h0, sem_h1,
             sem_resp):
    c = lax.axis_index("c")
    s = lax.axis_index("s")
    chunk = n_total // NS_
    rounds = chunk // P_
    half_lo = c * HALF_
    lane = jnp.arange(LANES_, dtype=jnp.int32)
    lane3 = lane * 3

    pltpu.sync_copy(gv_hbm.at[pl.ds(half_lo + s * TILEG_, TILEG_)],
                    subgrid.at[pl.ds(0, TILEG_)])
    subgrid[pl.ds(TILEG_, LANES_)] = jnp.zeros((LANES_,), jnp.float32)
    plsc.subcore_barrier()

    chunkbase = s * chunk

    def pos_copy(r, pb):
        return pltpu.make_async_copy(
            pos_hbm.at[pl.ds((chunkbase + r * P_) * 3, P_ * 3)],
            posbuf.at[pl.ds(pb * (P_ * 3), P_ * 3)], sem_pos)

    def imp_copy(r):
        return pltpu.make_async_copy(
            imp_hbm.at[pl.ds(chunkbase + r * P_, P_)],
            sp_val.at[pl.ds((r % 3) * RP_ + s * P_, P_)], sem_imp)

    pos_copy(0, 0).start()
    imp_copy(0).start()

    @pl.loop(0, rounds)
    def _p1(r):
        pb = r % 2
        par = (r % 2) * RP_
        pos_copy(r, pb).wait()

        @plsc.parallel_loop(0, NVEC_, unroll=8)
        def _vec(j):
            off = pb * (P_ * 3) + j * (3 * LANES_)
            x = plsc.load_gather(posbuf, [lane3 + off])
            y = plsc.load_gather(posbuf, [lane3 + (off + 1)])
            z = plsc.load_gather(posbuf, [lane3 + (off + 2)])
            lin = (_voxelize(x) * RES_ + _voxelize(y)) * RES_ + _voxelize(z)
            keep = (lin >= half_lo) & (lin < half_lo + HALF_)
            prodidx[pl.ds(j * LANES_, LANES_)] = (
                jnp.where(keep, lin - half_lo, HALF_))

        cp_scr = pltpu.async_copy(
            prodidx, idxscr_hbm.at[c, pl.ds(r * RP_ + s * P_, P_)], sem_scr)

        @pl.when(r + 1 < rounds)
        def _prefetch():
            pos_copy(r + 1, 1 - pb).start()
            imp_copy(r + 1).start()

        pltpu.sync_copy(prodidx, sp_idx.at[pl.ds(par + s * P_, P_)])
        cp_scr.wait()
        imp_copy(r).wait()
        plsc.subcore_barrier()

        cp_ci = pltpu.async_copy(sp_idx.at[pl.ds(par, RP_)], considx, sem_ci)
        vpar = (r % 3) * RP_
        cp_cv = pltpu.async_copy(sp_val.at[pl.ds(vpar, HB_)],
                                 consval, sem_cv)
        cp_ci.wait()
        cp_cv.wait()

        def add_half(h):
            @plsc.parallel_loop(0, HV_, unroll=8)
            def _add(j):
                vidx = considx[pl.ds(h * HB_ + j * LANES_, LANES_)]
                vval = consval[pl.ds(j * LANES_, LANES_)]
                own = (vidx >> 16) == s
                plsc.addupdate_scatter(subgrid, [vidx & (TILEG_ - 1)], vval,
                                       mask=own)

        add_half(0)
        pltpu.sync_copy(sp_val.at[pl.ds(vpar + HB_, HB_)], consval)
        add_half(1)

    plsc.subcore_barrier()

    @pl.loop(0, rounds)
    def _p2(r):
        base = chunkbase + r * P_
        cp_h0 = pltpu.async_copy(
            idxscr_hbm.at[c, pl.ds(r * RP_, HB_)],
            considx.at[pl.ds(0, HB_)], sem_h0)
        cp_h1 = pltpu.async_copy(
            idxscr_hbm.at[c, pl.ds(r * RP_ + HB_, HB_)],
            considx.at[pl.ds(HB_, HB_)], sem_h1)

        def answer_half(h):
            @plsc.parallel_loop(0, HV_, unroll=8)
            def _ans(j):
                vidx = considx[pl.ds(h * HB_ + j * LANES_, LANES_)]
                own = (vidx >> 16) == s
                lidx = jnp.where(own, vidx & (TILEG_ - 1), TILEG_)
                consval[pl.ds(j * LANES_, LANES_)] = (
                    plsc.load_gather(subgrid, [lidx]))
            return [pltpu.async_copy(
                consval.at[pl.ds(p * P_, P_)],
                sp_resp.at[pl.ds(((h * 8 + p) * NS_ + s) * P_, P_)],
                sem_resp) for p in range(8)]

        cp_h0.wait()
        resp_cp = answer_half(0)
        cp_h1.wait()
        for cp in resp_cp:
            cp.wait()
        resp_cp = answer_half(1)
        for cp in resp_cp:
            cp.wait()
        plsc.subcore_barrier()

        for h in range(2):
            pltpu.sync_copy(
                sp_resp.at[pl.ds((s * NS_ + h * 8) * P_, 8 * P_)],
                resp8.at[pl.ds(0, 8 * P_)])

            @plsc.parallel_loop(0, NVEC_, unroll=8)
            def _comb(j):
                slot = lane + j * LANES_
                vidx = considx[pl.ds(s * P_ + j * LANES_, LANES_)]
                dloc = (vidx >> 16) - h * 8
                inh = (dloc >= 0) & (dloc < 8)
                rv = plsc.load_gather(
                    resp8, [jnp.where(inh, dloc, 8) * P_ + slot])
                val = jnp.where(inh, rv, jnp.float32(0.0))
                if h == 0:
                    outbuf[pl.ds(j * LANES_, LANES_)] = val
                else:
                    prev = outbuf[pl.ds(j * LANES_, LANES_)]
                    outbuf[pl.ds(j * LANES_, LANES_)] = prev + val

        pltpu.sync_copy(outbuf, out_hbm.at[c, pl.ds(base, P_)])
        plsc.subcore_barrier()


def kernel(positions, importances, grid_values):
    n = positions.shape[0]
    pos_flat = positions.reshape(-1)
    imp_flat = importances.reshape(-1)
    gv = grid_values.reshape(-1)
    mesh = plsc.VectorSubcoreMesh(core_axis_name="c", subcore_axis_name="s",
                                  num_cores=NC_, num_subcores=NS_)
    out, _ = pl.kernel(
        functools.partial(_sc_body, n),
        out_type=(jax.ShapeDtypeStruct((NC_, n), jnp.float32),
                  jax.ShapeDtypeStruct((NC_, n), jnp.int32)),
        mesh=mesh,
        compiler_params=pltpu.CompilerParams(needs_layout_passes=False),
        scratch_types=[
            pltpu.VMEM((2 * P_ * 3,), jnp.float32),
            pltpu.VMEM((P_,), jnp.int32),
            pltpu.VMEM((RP_,), jnp.int32),
            pltpu.VMEM((HB_,), jnp.float32),
            pltpu.VMEM((9 * P_,), jnp.float32),
            pltpu.VMEM((P_,), jnp.float32),
            pltpu.VMEM((TILEG_ + LANES_,), jnp.float32),
            pltpu.VMEM_SHARED((2 * RP_,), jnp.int32),
            pltpu.VMEM_SHARED((3 * RP_,), jnp.float32),
            pltpu.VMEM_SHARED((NS_ * RP_,), jnp.float32),
            pltpu.SemaphoreType.DMA,
            pltpu.SemaphoreType.DMA,
            pltpu.SemaphoreType.DMA,
            pltpu.SemaphoreType.DMA,
            pltpu.SemaphoreType.DMA,
            pltpu.SemaphoreType.DMA,
            pltpu.SemaphoreType.DMA,
            pltpu.SemaphoreType.DMA,
        ],
    )(pos_flat, imp_flat, gv)
    return (out[0] + out[1]).reshape(n, 1)


# speedup vs baseline: 1.6951x; 1.0004x over previous
import functools

import jax
import jax.numpy as jnp
from jax import lax
from jax.experimental import pallas as pl
from jax.experimental.pallas import tpu as pltpu
from jax.experimental.pallas import tpu_sc as plsc

RES_ = 128
GRID_ = RES_ * RES_ * RES_
NC_ = 2
NS_ = 16
LANES_ = 16
HALF_ = GRID_ // NC_
TILEG_ = HALF_ // NS_
P_ = 1024
NVEC_ = P_ // LANES_
RP_ = NS_ * P_
HB_ = RP_ // 2
HV_ = HB_ // LANES_


def _voxelize(p):
    u = (p + 1.0) * jnp.float32(0.5) * jnp.float32(RES_)
    i = u.astype(jnp.int32)
    return jnp.clip(i, 0, RES_ - 1)


def _sc_body(n_total, pos_hbm, imp_hbm, gv_hbm, out_hbm, idxscr_hbm,
             posbuf, prodidx, considx, consval, resp8, outbuf, subgrid,
             sp_idx, sp_val, sp_resp,
             sem_pos, sem_imp, sem_scr, sem_ci, sem_cv, sem_h0, sem_h1,
             sem_resp):
    c = lax.axis_index("c")
    s = lax.axis_index("s")
    chunk = n_total // NS_
    rounds = chunk // P_
    half_lo = c * HALF_
    lane = jnp.arange(LANES_, dtype=jnp.int32)
    lane3 = lane * 3

    pltpu.sync_copy(gv_hbm.at[pl.ds(half_lo + s * TILEG_, TILEG_)],
                    subgrid.at[pl.ds(0, TILEG_)])
    subgrid[pl.ds(TILEG_, LANES_)] = jnp.zeros((LANES_,), jnp.float32)
    plsc.subcore_barrier()

    chunkbase = s * chunk

    def pos_copy(r, pb):
        return pltpu.make_async_copy(
            pos_hbm.at[pl.ds((chunkbase + r * P_) * 3, P_ * 3)],
            posbuf.at[pl.ds(pb * (P_ * 3), P_ * 3)], sem_pos)

    def imp_copy(r):
        return pltpu.make_async_copy(
            imp_hbm.at[pl.ds(chunkbase + r * P_, P_)],
            sp_val.at[pl.ds((r % 3) * RP_ + s * P_, P_)], sem_imp)

    pos_copy(0, 0).start()
    imp_copy(0).start()

    @pl.loop(0, rounds)
    def _p1(r):
        pb = r % 2
        par = (r % 2) * RP_
        pos_copy(r, pb).wait()

        @plsc.parallel_loop(0, NVEC_, unroll=8)
        def _vec(j):
            off = pb * (P_ * 3) + j * (3 * LANES_)
            x = plsc.load_gather(posbuf, [lane3 + off])
            y = plsc.load_gather(posbuf, [lane3 + (off + 1)])
            z = plsc.load_gather(posbuf, [lane3 + (off + 2)])
            lin = (_voxelize(x) * RES_ + _voxelize(y)) * RES_ + _voxelize(z)
            keep = (lin >= half_lo) & (lin < half_lo + HALF_)
            prodidx[pl.ds(j * LANES_, LANES_)] = (
                jnp.where(keep, lin - half_lo, HALF_))

        cp_scr = pltpu.async_copy(
            prodidx, idxscr_hbm.at[c, pl.ds(r * RP_ + s * P_, P_)], sem_scr)

        @pl.when(r + 1 < rounds)
        def _prefetch():
            pos_copy(r + 1, 1 - pb).start()
            imp_copy(r + 1).start()

        pltpu.sync_copy(prodidx, sp_idx.at[pl.ds(par + s * P_, P_)])
        cp_scr.wait()
        imp_copy(r).wait()
        plsc.subcore_barrier()

        cp_ci = pltpu.async_copy(sp_idx.at[pl.ds(par, RP_)], considx, sem_ci)
        vpar = (r % 3) * RP_
        cp_cv = pltpu.async_copy(sp_val.at[pl.ds(vpar, HB_)],
                                 consval, sem_cv)
        cp_ci.wait()
        cp_cv.wait()

        def add_half(h):
            @plsc.parallel_loop(0, HV_, unroll=16)
            def _add(j):
                vidx = considx[pl.ds(h * HB_ + j * LANES_, LANES_)]
                vval = consval[pl.ds(j * LANES_, LANES_)]
                own = (vidx >> 16) == s
                plsc.addupdate_scatter(subgrid, [vidx & (TILEG_ - 1)], vval,
                                       mask=own)

        add_half(0)
        pltpu.sync_copy(sp_val.at[pl.ds(vpar + HB_, HB_)], consval)
        add_half(1)

    plsc.subcore_barrier()

    @pl.loop(0, rounds)
    def _p2(r):
        base = chunkbase + r * P_
        cp_h0 = pltpu.async_copy(
            idxscr_hbm.at[c, pl.ds(r * RP_, HB_)],
            considx.at[pl.ds(0, HB_)], sem_h0)
        cp_h1 = pltpu.async_copy(
            idxscr_hbm.at[c, pl.ds(r * RP_ + HB_, HB_)],
            considx.at[pl.ds(HB_, HB_)], sem_h1)

        def answer_half(h):
            @plsc.parallel_loop(0, HV_, unroll=16)
            def _ans(j):
                vidx = considx[pl.ds(h * HB_ + j * LANES_, LANES_)]
                own = (vidx >> 16) == s
                lidx = jnp.where(own, vidx & (TILEG_ - 1), TILEG_)
                consval[pl.ds(j * LANES_, LANES_)] = (
                    plsc.load_gather(subgrid, [lidx]))
            return [pltpu.async_copy(
                consval.at[pl.ds(p * P_, P_)],
                sp_resp.at[pl.ds(((h * 8 + p) * NS_ + s) * P_, P_)],
                sem_resp) for p in range(8)]

        cp_h0.wait()
        resp_cp = answer_half(0)
        cp_h1.wait()
        for cp in resp_cp:
            cp.wait()
        resp_cp = answer_half(1)
        for cp in resp_cp:
            cp.wait()
        plsc.subcore_barrier()

        for h in range(2):
            pltpu.sync_copy(
                sp_resp.at[pl.ds((s * NS_ + h * 8) * P_, 8 * P_)],
                resp8.at[pl.ds(0, 8 * P_)])

            @plsc.parallel_loop(0, NVEC_, unroll=8)
            def _comb(j):
                slot = lane + j * LANES_
                vidx = considx[pl.ds(s * P_ + j * LANES_, LANES_)]
                dloc = (vidx >> 16) - h * 8
                inh = (dloc >= 0) & (dloc < 8)
                rv = plsc.load_gather(
                    resp8, [jnp.where(inh, dloc, 8) * P_ + slot])
                val = jnp.where(inh, rv, jnp.float32(0.0))
                if h == 0:
                    outbuf[pl.ds(j * LANES_, LANES_)] = val
                else:
                    prev = outbuf[pl.ds(j * LANES_, LANES_)]
                    outbuf[pl.ds(j * LANES_, LANES_)] = prev + val

        pltpu.sync_copy(outbuf, out_hbm.at[c, pl.ds(base, P_)])
        plsc.subcore_barrier()


def kernel(positions, importances, grid_values):
    n = positions.shape[0]
    pos_flat = positions.reshape(-1)
    imp_flat = importances.reshape(-1)
    gv = grid_values.reshape(-1)
    mesh = plsc.VectorSubcoreMesh(core_axis_name="c", subcore_axis_name="s",
                                  num_cores=NC_, num_subcores=NS_)
    out, _ = pl.kernel(
        functools.partial(_sc_body, n),
        out_type=(jax.ShapeDtypeStruct((NC_, n), jnp.float32),
                  jax.ShapeDtypeStruct((NC_, n), jnp.int32)),
        mesh=mesh,
        compiler_params=pltpu.CompilerParams(needs_layout_passes=False),
        scratch_types=[
            pltpu.VMEM((2 * P_ * 3,), jnp.float32),
            pltpu.VMEM((P_,), jnp.int32),
            pltpu.VMEM((RP_,), jnp.int32),
            pltpu.VMEM((HB_,), jnp.float32),
            pltpu.VMEM((9 * P_,), jnp.float32),
            pltpu.VMEM((P_,), jnp.float32),
            pltpu.VMEM((TILEG_ + LANES_,), jnp.float32),
            pltpu.VMEM_SHARED((2 * RP_,), jnp.int32),
            pltpu.VMEM_SHARED((3 * RP_,), jnp.float32),
            pltpu.VMEM_SHARED((NS_ * RP_,), jnp.float32),
            pltpu.SemaphoreType.DMA,
            pltpu.SemaphoreType.DMA,
            pltpu.SemaphoreType.DMA,
            pltpu.SemaphoreType.DMA,
            pltpu.SemaphoreType.DMA,
            pltpu.SemaphoreType.DMA,
            pltpu.SemaphoreType.DMA,
            pltpu.SemaphoreType.DMA,
        ],
    )(pos_flat, imp_flat, gv)
    return (out[0] + out[1]).reshape(n, 1)
